# baseline (speedup 1.0000x reference)
import functools

import jax
import jax.numpy as jnp
from jax import lax
from jax.experimental import pallas as pl
from jax.experimental.pallas import tpu as pltpu
from jax.experimental.pallas import tpu_sc as plsc

N_NODES = 10000
N_EDGES = 160000
UNITS = 32
CUTOFF = 5.0

NC = 2
NS = 16
CH = 40
EPW = N_EDGES // NS
NPT = N_NODES // NS
BE = 3200
BN = 1000


def _edge_kernel(ea_ref, bd_ref, bv_ref, wc_ref, bc_ref, we3_ref, be3_ref,
                 wb_ref, ef_ref):
    ea = ea_ref[...]

    def mm(x, w):
        return lax.dot_general(x, w, (((1,), (1,)), ((), ())),
                               preferred_element_type=jnp.float32,
                               precision=lax.Precision.HIGHEST)

    wb_ref[...] = mm(ea, wc_ref[...]) + bc_ref[...]

    r = bd_ref[...]
    c = jnp.where(r <= CUTOFF, 0.5 * (jnp.cos(jnp.pi * r / CUTOFF) + 1.0), 0.0)

    v = bv_ref[...]
    inv = 1.0 / jnp.sqrt(jnp.sum(v * v, axis=0, keepdims=True))
    ev = v * inv
    sq = ev * ev
    evr = jnp.concatenate([ev[1:], ev[:1]], axis=0)
    cr = ev * evr
    comp = jnp.concatenate([ev, sq, cr, c, ev, ev], axis=0)
    basc = comp.T

    wb_ref[:, 48:64] = basc
    wb_ref[:, 112:128] = basc

    ef_ref[...] = mm(ea, we3_ref[...]) + be3_ref[...]


def _edge_precompute(edge_attr, bond_dist, bond_vec, Wd1, bd1, Wd2, bd2,
                     Wd3, bd3, W_emb3, b_emb3):
    z16 = jnp.zeros((16, 32), jnp.float32)
    wcomb = jnp.concatenate(
        [Wd1[:16], Wd2[:16], Wd3[:16], z16,
         Wd1[16:], Wd2[16:], Wd3[16:], z16], axis=0)
    zb = jnp.zeros((16,), jnp.float32)
    bcomb = jnp.concatenate(
        [bd1[:16], bd2[:16], bd3[:16], zb,
         bd1[16:], bd2[16:], bd3[16:], zb], axis=0)
    grid = (N_EDGES // BE,)
    return pl.pallas_call(
        _edge_kernel,
        grid=grid,
        in_specs=[
            pl.BlockSpec((BE, 32), lambda i: (i, 0)),
            pl.BlockSpec((1, BE), lambda i: (0, i)),
            pl.BlockSpec((3, BE), lambda i: (0, i)),
            pl.BlockSpec((128, 32), lambda i: (0, 0)),
            pl.BlockSpec((1, 128), lambda i: (0, 0)),
            pl.BlockSpec((32, 32), lambda i: (0, 0)),
            pl.BlockSpec((1, 32), lambda i: (0, 0)),
        ],
        out_specs=[
            pl.BlockSpec((BE, 128), lambda i: (i, 0)),
            pl.BlockSpec((BE, 32), lambda i: (i, 0)),
        ],
        out_shape=[
            jax.ShapeDtypeStruct((N_EDGES, 128), jnp.float32),
            jax.ShapeDtypeStruct((N_EDGES, 32), jnp.float32),
        ],
    )(edge_attr, bond_dist[None, :], bond_vec.T, wcomb, bcomb[None, :],
      W_emb3, b_emb3[None, :])


def _node_kernel(nt_ref, emb_ref, wa_ref, wb_ref, b2_ref, u_ref, v_ref):
    nt = nt_ref[...]
    iota = lax.broadcasted_iota(jnp.int32, (BN, 128), 1)
    oh = (nt == iota).astype(jnp.float32)

    def mm_t(x, w):
        return lax.dot_general(x, w, (((1,), (1,)), ((), ())),
                               preferred_element_type=jnp.float32,
                               precision=lax.Precision.HIGHEST)

    def mm(x, w):
        return lax.dot_general(x, w, (((1,), (0,)), ((), ())),
                               preferred_element_type=jnp.float32,
                               precision=lax.Precision.HIGHEST)

    ma = mm_t(emb_ref[...], wa_ref[...])
    mb = mm_t(emb_ref[...], wb_ref[...])
    u = mm(oh, ma)
    v = mm(oh, mb) + b2_ref[...]
    u_ref[0] = u[:, :16]
    u_ref[1] = u[:, 16:]
    v_ref[0] = v[:, :16]
    v_ref[1] = v[:, 16:]


def _node_precompute(node_type, emb_table, W_emb2, b_emb2):
    emb_pad = jnp.zeros((128, 32), jnp.float32).at[:emb_table.shape[0]].set(
        emb_table)
    wa = W_emb2[:, :UNITS]
    wb = W_emb2[:, UNITS:]
    grid = (N_NODES // BN,)
    full = lambda s: pl.BlockSpec(s, lambda i: tuple(0 for _ in s))
    return pl.pallas_call(
        _node_kernel,
        grid=grid,
        in_specs=[
            pl.BlockSpec((BN, 1), lambda i: (i, 0)),
            full((128, 32)), full((32, 32)), full((32, 32)), full((1, 32)),
        ],
        out_specs=[
            pl.BlockSpec((2, BN, 16), lambda i: (0, i, 0)),
            pl.BlockSpec((2, BN, 16), lambda i: (0, i, 0)),
        ],
        out_shape=[
            jax.ShapeDtypeStruct((2, N_NODES, 16), jnp.float32),
            jax.ShapeDtypeStruct((2, N_NODES, 16), jnp.float32),
        ],
    )(node_type[:, None], emb_pad, wa, wb, b_emb2[None, :])


def _sc_scatter(edge_index, WB, U2, V2):
    mesh = plsc.VectorSubcoreMesh(core_axis_name="c", subcore_axis_name="s")
    n_chunks = EPW // CH

    vset = lambda: [pltpu.VMEM((2, CH), jnp.int32),
                    pltpu.VMEM((CH, 64), jnp.float32),
                    pltpu.VMEM((CH, 16), jnp.float32),
                    pltpu.VMEM((CH, 16), jnp.float32)]

    @functools.partial(
        pl.kernel,
        out_type=[jax.ShapeDtypeStruct((NC, N_NODES, 128), jnp.float32),
                  jax.ShapeDtypeStruct((NC, N_NODES, 32), jnp.float32)],
        mesh=mesh,
        compiler_params=pltpu.CompilerParams(use_tc_tiling_on_sc=False),
        scratch_types=[
            pltpu.VMEM_SHARED((N_NODES, 160), jnp.float32),
            vset(), vset(),
            pltpu.VMEM((CH, 160), jnp.float32),
            pltpu.VMEM((CH, 160), jnp.float32),
            pltpu.VMEM((CH,), jnp.int32),
            pltpu.VMEM((CH,), jnp.int32),
            pltpu.SemaphoreType.DMA, pltpu.SemaphoreType.DMA,
            pltpu.SemaphoreType.DMA, pltpu.SemaphoreType.DMA,
            pltpu.SemaphoreType.DMA, pltpu.SemaphoreType.DMA,
            pltpu.SemaphoreType.DMA, pltpu.SemaphoreType.DMA,
        ],
    )
    def sc_fn(ei, wb, u2, v2, out, out2, acc, set0, set1, pay0, pay1,
              dsts0, dsts1, sl0, sl1, sg0, sg1, ss0, ss1, sd0, sd1):
        c = lax.axis_index("c")
        s = lax.axis_index("s")
        base_n = s * NPT
        sets = (set0, set1)
        pays = (pay0, pay1)
        dstss = (dsts0, dsts1)
        sem_l = (sl0, sl1)
        sem_g = (sg0, sg1)
        sem_s = (ss0, ss1)
        sem_d = (sd0, sd1)

        def zrow(i, _):
            for k in range(10):
                pay0[i, pl.ds(16 * k, 16)] = jnp.zeros((16,), jnp.float32)
            return 0
        lax.fori_loop(0, CH, zrow, 0)

        def zcopy(j, _):
            pltpu.sync_copy(pay0, acc.at[pl.ds(base_n + CH * j, CH)])
            return 0
        lax.fori_loop(0, NPT // CH, zcopy, 0)
        rem = NPT - (NPT // CH) * CH
        if rem:
            pltpu.sync_copy(pay0.at[pl.ds(0, rem)],
                            acc.at[pl.ds(base_n + (NPT // CH) * CH, rem)])
        plsc.subcore_barrier()

        def e_of(i):
            return s * EPW + jnp.minimum(i, n_chunks - 1) * CH

        def fire_linear(i, p):
            idx, wc, _, _ = sets[p]
            e0 = e_of(i)
            pltpu.async_copy(ei.at[:, pl.ds(e0, CH)], idx, sem_l[p])
            pltpu.async_copy(wb.at[pl.ds(e0, CH), pl.ds(64 * c, 64)],
                             wc, sem_l[p])

        def wait_linear(p):
            idx, wc, _, _ = sets[p]
            e0 = s * EPW
            pltpu.make_async_copy(ei.at[:, pl.ds(e0, CH)], idx, sem_l[p]).wait()
            pltpu.make_async_copy(wb.at[pl.ds(e0, CH), pl.ds(0, 64)],
                                  wc, sem_l[p]).wait()

        def fire_gathers(p):
            idx, _, ur, vr = sets[p]
            pltpu.async_copy(u2.at[c].at[idx.at[0]], ur, sem_g[p])
            pltpu.async_copy(v2.at[c].at[idx.at[1]], vr, sem_g[p])

        def wait_gathers(p):
            idx, _, ur, vr = sets[p]
            pltpu.make_async_copy(u2.at[c].at[idx.at[0]], ur, sem_g[p]).wait()
            pltpu.make_async_copy(v2.at[c].at[idx.at[1]], vr, sem_g[p]).wait()

        def fire_scatter(p):
            pltpu.async_copy(pays[p], acc.at[dstss[p]], sem_s[p], add=True)

        def wait_scatter(p):
            pltpu.make_async_copy(pays[p], acc.at[dstss[p]], sem_s[p]).wait()

        def run_chunk(i, p, first):
            q = 1 - p
            idx, wc, ur, vr = sets[p]
            pay = pays[p]
            wait_linear(q)
            fire_gathers(q)
            wait_gathers(p)
            if not first:
                wait_scatter(p)
            e0 = s * EPW + i * CH
            pltpu.async_copy(ei.at[1, pl.ds(e0, CH)], dstss[p], sem_d[p])

            def edge_body(e, _):
                bb = wc[e, pl.ds(48, 16)]
                zc = (ur[e, :] + vr[e, :]) * bb[9]
                g1 = zc * wc[e, pl.ds(0, 16)]
                g2 = zc * wc[e, pl.ds(16, 16)]
                g3 = zc * wc[e, pl.ds(32, 16)]
                pay[e, pl.ds(0, 16)] = g1
                for d in range(3):
                    pay[e, pl.ds(16 + 16 * d, 16)] = g2 * bb[d]
                for k in range(6):
                    pay[e, pl.ds(64 + 16 * k, 16)] = g3 * bb[3 + k]
                return 0

            def edge_pair(j, _):
                edge_body(2 * j, 0)
                edge_body(2 * j + 1, 0)
                return 0
            lax.fori_loop(0, CH // 2, edge_pair, 0)

            pltpu.make_async_copy(ei.at[1, pl.ds(e0, CH)], dstss[p],
                                  sem_d[p]).wait()
            fire_scatter(p)
            fire_linear(i + 2, p)

        fire_linear(0, 0)
        fire_linear(1, 1)
        wait_linear(0)
        fire_gathers(0)

        run_chunk(0, 0, True)
        run_chunk(1, 1, True)

        def body2(k, _):
            run_chunk(2 * k, 0, False)
            run_chunk(2 * k + 1, 1, False)
            return 0
        lax.fori_loop(1, n_chunks // 2, body2, 0)

        wait_linear(1)
        wait_gathers(0)
        wait_scatter(0)
        wait_scatter(1)
        plsc.subcore_barrier()

        def drain(j, _):
            r0 = base_n + CH * j
            pltpu.sync_copy(acc.at[pl.ds(r0, CH)], pay0)
            pltpu.sync_copy(pay0.at[pl.ds(0, CH), pl.ds(0, 128)],
                            out.at[c, pl.ds(r0, CH)])
            pltpu.sync_copy(pay0.at[pl.ds(0, CH), pl.ds(128, 32)],
                            out2.at[c, pl.ds(r0, CH)])
            return 0
        lax.fori_loop(0, NPT // CH, drain, 0)
        if rem:
            r0 = base_n + (NPT // CH) * CH
            pltpu.sync_copy(acc.at[pl.ds(r0, rem)], pay0.at[pl.ds(0, rem)])
            pltpu.sync_copy(pay0.at[pl.ds(0, rem), pl.ds(0, 128)],
                            out.at[c, pl.ds(r0, rem)])
            pltpu.sync_copy(pay0.at[pl.ds(0, rem), pl.ds(128, 32)],
                            out2.at[c, pl.ds(r0, rem)])

    return sc_fn(edge_index, WB, U2, V2)


def _finish_kernel(ga0_ref, ga1_ref, gb0_ref, gb1_ref, lng_ref, lnb_ref,
                   ws0_ref, bs0_ref, ws1_ref, bs1_ref,
                   wa0_ref, wa1_ref, wb0_ref, wb1_ref, *o_refs):
    ga = (ga0_ref[...], ga1_ref[...])
    gb = (gb0_ref[...], gb1_ref[...])

    def half_norm(a, b):
        q = a * a
        s = lambda k: q[:, 16 * k:16 * k + 16]
        qb = b * b
        trp = a[:, 64:80] + a[:, 80:96] + a[:, 96:112]
        return (3.0 * s(0) + 2.0 * (s(1) + s(2) + s(3))
                + s(4) + s(5) + s(6)
                + 2.0 * (s(7) + qb[:, 0:16] + qb[:, 16:32])
                - trp * trp / 3.0)

    nrm = jnp.concatenate([half_norm(ga[0], gb[0]),
                           half_norm(ga[1], gb[1])], axis=1)
    mu = jnp.mean(nrm, axis=1, keepdims=True)
    var = jnp.mean((nrm - mu) ** 2, axis=1, keepdims=True)
    nrm = (nrm - mu) / jnp.sqrt(var + 1e-5) * lng_ref[...] + lnb_ref[...]

    def mm_t(x, w):
        return lax.dot_general(x, w, (((1,), (1,)), ((), ())),
                               preferred_element_type=jnp.float32,
                               precision=lax.Precision.HIGHEST)

    h = mm_t(nrm, ws0_ref[...]) + bs0_ref[...]
    h = h * jax.nn.sigmoid(h)
    h = mm_t(h, ws1_ref[...]) + bs1_ref[...]
    h = h * jax.nn.sigmoid(h)
    n0 = h[:, 0:32]
    n1 = h[:, 32:64]
    n2 = h[:, 64:96]

    pb = (mm_t(ga[0], wa0_ref[...]) + mm_t(ga[1], wa1_ref[...]))
    pb2 = (mm_t(gb[0], wb0_ref[...]) + mm_t(gb[1], wb1_ref[...]))
    A0 = pb[:, 0:32]
    w0 = pb[:, 32:64]
    w1 = pb[:, 64:96]
    w2 = pb[:, 96:128]
    Pp = [pb[:, 128:160], pb[:, 160:192], pb[:, 192:224], pb[:, 224:256],
          pb2[:, 0:32], pb2[:, 32:64]]
    t3 = (Pp[0] + Pp[1] + Pp[2]) / 3.0

    diag = n0 * A0
    o_refs[0][...] = diag + n2 * (Pp[0] - t3)
    o_refs[1][...] = n2 * Pp[3] - n1 * w2
    o_refs[2][...] = n2 * Pp[5] + n1 * w1
    o_refs[3][...] = n2 * Pp[3] + n1 * w2
    o_refs[4][...] = diag + n2 * (Pp[1] - t3)
    o_refs[5][...] = n2 * Pp[4] - n1 * w0
    o_refs[6][...] = n2 * Pp[5] - n1 * w1
    o_refs[7][...] = n2 * Pp[4] + n1 * w0
    o_refs[8][...] = diag + n2 * (Pp[2] - t3)


def _node_finish(GhA, GhB, ln_g, ln_b, Ws0, bs0, Ws1, bs1, Wt0, Wt1, Wt2):
    perm = jnp.asarray([3 * c + k for k in range(3) for c in range(32)],
                       dtype=jnp.int32)
    ws1p = Ws1[perm, :]
    bs1p = bs1[perm]
    comps_a = [Wt0, Wt1, Wt1, Wt1, Wt2, Wt2, Wt2, Wt2]
    wa = [jnp.zeros((256, 128), jnp.float32) for _ in range(2)]
    wb = [jnp.zeros((64, 32), jnp.float32) for _ in range(2)]
    for h in range(2):
        for k, Wk in enumerate(comps_a):
            wa[h] = wa[h].at[32 * k:32 * k + 32, 16 * k:16 * k + 16].set(
                Wk[:, 16 * h:16 * h + 16])
        for k in range(2):
            wb[h] = wb[h].at[32 * k:32 * k + 32, 16 * k:16 * k + 16].set(
                Wt2[:, 16 * h:16 * h + 16])
    grid = (N_NODES // BN,)
    full = lambda s: pl.BlockSpec(s, lambda i: tuple(0 for _ in s))
    outs = pl.pallas_call(
        _finish_kernel,
        grid=grid,
        in_specs=[
            pl.BlockSpec((BN, 128), lambda i: (i, 0)),
            pl.BlockSpec((BN, 128), lambda i: (i, 0)),
            pl.BlockSpec((BN, 32), lambda i: (i, 0)),
            pl.BlockSpec((BN, 32), lambda i: (i, 0)),
            full((1, 32)), full((1, 32)), full((64, 32)), full((1, 64)),
            full((96, 64)), full((1, 96)),
            full((256, 128)), full((256, 128)), full((64, 32)), full((64, 32)),
        ],
        out_specs=[pl.BlockSpec((BN, 32), lambda i: (i, 0))] * 9,
        out_shape=[jax.ShapeDtypeStruct((N_NODES, 32), jnp.float32)] * 9,
    )(GhA[0], GhA[1], GhB[0], GhB[1], ln_g[None, :], ln_b[None, :],
      Ws0, bs0[None, :], ws1p, bs1p[None, :], wa[0], wa[1], wb[0], wb[1])
    return jnp.stack(outs, axis=-1).reshape(N_NODES, UNITS, 3, 3)


def kernel(node_type, edge_index, edge_attr, bond_dist, bond_vec, emb_table,
           Wd1, bd1, Wd2, bd2, Wd3, bd3, W_emb2, b_emb2, W_emb3, b_emb3,
           Wt0, Wt1, Wt2, Ws0, bs0, Ws1, bs1, ln_g, ln_b):
    WB, edge_feat = _edge_precompute(
        edge_attr, bond_dist, bond_vec, Wd1, bd1, Wd2, bd2, Wd3, bd3,
        W_emb3, b_emb3)
    U2, V2 = _node_precompute(node_type, emb_table, W_emb2, b_emb2)
    GhA, GhB = _sc_scatter(edge_index, WB, U2, V2)
    X = _node_finish(GhA, GhB, ln_g, ln_b, Ws0, bs0, Ws1, bs1, Wt0, Wt1, Wt2)
    return X, edge_feat


# speedup vs baseline: 1.0192x; 1.0192x over previous
import functools

import jax
import jax.numpy as jnp
from jax import lax
from jax.experimental import pallas as pl
from jax.experimental.pallas import tpu as pltpu
from jax.experimental.pallas import tpu_sc as plsc

N_NODES = 10000
N_EDGES = 160000
UNITS = 32
CUTOFF = 5.0

NC = 2
NS = 16
CH = 40
EPW = N_EDGES // NS
NPT = N_NODES // NS
BE = 3200
BN = 1000


def _edge_kernel(ea_ref, bd_ref, bv_ref, wc_ref, bc_ref, we3_ref, be3_ref,
                 wb_ref, ef_ref):
    ea = ea_ref[...]

    def mm(x, w):
        return lax.dot_general(x, w, (((1,), (1,)), ((), ())),
                               preferred_element_type=jnp.float32,
                               precision=lax.Precision.HIGHEST)

    wb_ref[...] = mm(ea, wc_ref[...]) + bc_ref[...]

    r = bd_ref[...]
    c = jnp.where(r <= CUTOFF, 0.5 * (jnp.cos(jnp.pi * r / CUTOFF) + 1.0), 0.0)

    v = bv_ref[...]
    inv = 1.0 / jnp.sqrt(jnp.sum(v * v, axis=0, keepdims=True))
    ev = v * inv
    sq = ev * ev
    evr = jnp.concatenate([ev[1:], ev[:1]], axis=0)
    cr = ev * evr
    comp = jnp.concatenate([ev, sq, cr, c, ev, ev], axis=0)
    basc = comp.T

    wb_ref[:, 48:64] = basc
    wb_ref[:, 112:128] = basc

    ef_ref[...] = mm(ea, we3_ref[...]) + be3_ref[...]


def _edge_precompute(edge_attr, bond_dist, bond_vec, Wd1, bd1, Wd2, bd2,
                     Wd3, bd3, W_emb3, b_emb3):
    z16 = jnp.zeros((16, 32), jnp.float32)
    wcomb = jnp.concatenate(
        [Wd1[:16], Wd2[:16], Wd3[:16], z16,
         Wd1[16:], Wd2[16:], Wd3[16:], z16], axis=0)
    zb = jnp.zeros((16,), jnp.float32)
    bcomb = jnp.concatenate(
        [bd1[:16], bd2[:16], bd3[:16], zb,
         bd1[16:], bd2[16:], bd3[16:], zb], axis=0)
    grid = (N_EDGES // BE,)
    return pl.pallas_call(
        _edge_kernel,
        grid=grid,
        in_specs=[
            pl.BlockSpec((BE, 32), lambda i: (i, 0)),
            pl.BlockSpec((1, BE), lambda i: (0, i)),
            pl.BlockSpec((3, BE), lambda i: (0, i)),
            pl.BlockSpec((128, 32), lambda i: (0, 0)),
            pl.BlockSpec((1, 128), lambda i: (0, 0)),
            pl.BlockSpec((32, 32), lambda i: (0, 0)),
            pl.BlockSpec((1, 32), lambda i: (0, 0)),
        ],
        out_specs=[
            pl.BlockSpec((BE, 128), lambda i: (i, 0)),
            pl.BlockSpec((BE, 32), lambda i: (i, 0)),
        ],
        out_shape=[
            jax.ShapeDtypeStruct((N_EDGES, 128), jnp.float32),
            jax.ShapeDtypeStruct((N_EDGES, 32), jnp.float32),
        ],
    )(edge_attr, bond_dist[None, :], bond_vec.T, wcomb, bcomb[None, :],
      W_emb3, b_emb3[None, :])


def _node_kernel(nt_ref, emb_ref, wa_ref, wb_ref, b2_ref, u_ref, v_ref):
    nt = nt_ref[...]
    iota = lax.broadcasted_iota(jnp.int32, (BN, 128), 1)
    oh = (nt == iota).astype(jnp.float32)

    def mm_t(x, w):
        return lax.dot_general(x, w, (((1,), (1,)), ((), ())),
                               preferred_element_type=jnp.float32,
                               precision=lax.Precision.HIGHEST)

    def mm(x, w):
        return lax.dot_general(x, w, (((1,), (0,)), ((), ())),
                               preferred_element_type=jnp.float32,
                               precision=lax.Precision.HIGHEST)

    ma = mm_t(emb_ref[...], wa_ref[...])
    mb = mm_t(emb_ref[...], wb_ref[...])
    u = mm(oh, ma)
    v = mm(oh, mb) + b2_ref[...]
    u_ref[0] = u[:, :16]
    u_ref[1] = u[:, 16:]
    v_ref[0] = v[:, :16]
    v_ref[1] = v[:, 16:]


def _node_precompute(node_type, emb_table, W_emb2, b_emb2):
    emb_pad = jnp.zeros((128, 32), jnp.float32).at[:emb_table.shape[0]].set(
        emb_table)
    wa = W_emb2[:, :UNITS]
    wb = W_emb2[:, UNITS:]
    grid = (N_NODES // BN,)
    full = lambda s: pl.BlockSpec(s, lambda i: tuple(0 for _ in s))
    return pl.pallas_call(
        _node_kernel,
        grid=grid,
        in_specs=[
            pl.BlockSpec((BN, 1), lambda i: (i, 0)),
            full((128, 32)), full((32, 32)), full((32, 32)), full((1, 32)),
        ],
        out_specs=[
            pl.BlockSpec((2, BN, 16), lambda i: (0, i, 0)),
            pl.BlockSpec((2, BN, 16), lambda i: (0, i, 0)),
        ],
        out_shape=[
            jax.ShapeDtypeStruct((2, N_NODES, 16), jnp.float32),
            jax.ShapeDtypeStruct((2, N_NODES, 16), jnp.float32),
        ],
    )(node_type[:, None], emb_pad, wa, wb, b_emb2[None, :])


def _sc_scatter(edge_index, WB, U2, V2):
    mesh = plsc.VectorSubcoreMesh(core_axis_name="c", subcore_axis_name="s")
    n_chunks = EPW // CH

    vset = lambda: [pltpu.VMEM((CH,), jnp.int32),
                    pltpu.VMEM((CH,), jnp.int32),
                    pltpu.VMEM((CH, 64), jnp.float32),
                    pltpu.VMEM((CH, 16), jnp.float32),
                    pltpu.VMEM((CH, 16), jnp.float32)]

    @functools.partial(
        pl.kernel,
        out_type=[jax.ShapeDtypeStruct((NC, N_NODES, 128), jnp.float32),
                  jax.ShapeDtypeStruct((NC, N_NODES, 32), jnp.float32)],
        mesh=mesh,
        compiler_params=pltpu.CompilerParams(use_tc_tiling_on_sc=False),
        scratch_types=[
            pltpu.VMEM_SHARED((N_NODES, 160), jnp.float32),
            vset(), vset(),
            pltpu.VMEM((CH, 160), jnp.float32),
            pltpu.VMEM((CH, 160), jnp.float32),
            pltpu.VMEM((CH,), jnp.int32),
            pltpu.VMEM((CH,), jnp.int32),
            pltpu.SemaphoreType.DMA, pltpu.SemaphoreType.DMA,
            pltpu.SemaphoreType.DMA, pltpu.SemaphoreType.DMA,
            pltpu.SemaphoreType.DMA, pltpu.SemaphoreType.DMA,
            pltpu.SemaphoreType.DMA, pltpu.SemaphoreType.DMA,
        ],
    )
    def sc_fn(ei, wb, u2, v2, out, out2, acc, set0, set1, pay0, pay1,
              dsts0, dsts1, sl0, sl1, sg0, sg1, ss0, ss1, sd0, sd1):
        c = lax.axis_index("c")
        s = lax.axis_index("s")
        base_n = s * NPT
        sets = (set0, set1)
        pays = (pay0, pay1)
        dstss = (dsts0, dsts1)
        sem_l = (sl0, sl1)
        sem_g = (sg0, sg1)
        sem_s = (ss0, ss1)
        sem_d = (sd0, sd1)

        def zrow(i, _):
            for k in range(10):
                pay0[i, pl.ds(16 * k, 16)] = jnp.zeros((16,), jnp.float32)
            return 0
        lax.fori_loop(0, CH, zrow, 0)

        def zcopy(j, _):
            pltpu.sync_copy(pay0, acc.at[pl.ds(base_n + CH * j, CH)])
            return 0
        lax.fori_loop(0, NPT // CH, zcopy, 0)
        rem = NPT - (NPT // CH) * CH
        if rem:
            pltpu.sync_copy(pay0.at[pl.ds(0, rem)],
                            acc.at[pl.ds(base_n + (NPT // CH) * CH, rem)])
        plsc.subcore_barrier()

        def e_of(i):
            return s * EPW + jnp.minimum(i, n_chunks - 1) * CH

        def fire_linear(i, p):
            src_i, dst_i, wc, _, _ = sets[p]
            e0 = e_of(i)
            pltpu.async_copy(ei.at[0, pl.ds(e0, CH)], src_i, sem_l[p])
            pltpu.async_copy(ei.at[1, pl.ds(e0, CH)], dst_i, sem_l[p])
            pltpu.async_copy(wb.at[pl.ds(e0, CH), pl.ds(64 * c, 64)],
                             wc, sem_l[p])

        def wait_linear(p):
            src_i, dst_i, wc, _, _ = sets[p]
            e0 = s * EPW
            pltpu.make_async_copy(ei.at[0, pl.ds(e0, CH)], src_i, sem_l[p]).wait()
            pltpu.make_async_copy(ei.at[1, pl.ds(e0, CH)], dst_i, sem_l[p]).wait()
            pltpu.make_async_copy(wb.at[pl.ds(e0, CH), pl.ds(0, 64)],
                                  wc, sem_l[p]).wait()

        def fire_gathers(p):
            src_i, dst_i, _, ur, vr = sets[p]
            pltpu.async_copy(u2.at[c].at[src_i], ur, sem_g[p])
            pltpu.async_copy(v2.at[c].at[dst_i], vr, sem_g[p])

        def wait_gathers(p):
            src_i, dst_i, _, ur, vr = sets[p]
            pltpu.make_async_copy(u2.at[c].at[src_i], ur, sem_g[p]).wait()
            pltpu.make_async_copy(v2.at[c].at[dst_i], vr, sem_g[p]).wait()

        def fire_scatter(p):
            pltpu.async_copy(pays[p], acc.at[dstss[p]], sem_s[p], add=True)

        def wait_scatter(p):
            pltpu.make_async_copy(pays[p], acc.at[dstss[p]], sem_s[p]).wait()

        def run_chunk(i, p, first):
            q = 1 - p
            src_i, dst_i, wc, ur, vr = sets[p]
            pay = pays[p]
            wait_linear(q)
            fire_gathers(q)
            wait_gathers(p)
            if not first:
                wait_scatter(p)
            e0 = s * EPW + i * CH
            pltpu.async_copy(ei.at[1, pl.ds(e0, CH)], dstss[p], sem_d[p])

            def edge_body(e, _):
                bb = wc[e, pl.ds(48, 16)]
                zc = (ur[e, :] + vr[e, :]) * bb[9]
                g1 = zc * wc[e, pl.ds(0, 16)]
                g2 = zc * wc[e, pl.ds(16, 16)]
                g3 = zc * wc[e, pl.ds(32, 16)]
                pay[e, pl.ds(0, 16)] = g1
                for d in range(3):
                    pay[e, pl.ds(16 + 16 * d, 16)] = g2 * bb[d]
                for k in range(6):
                    pay[e, pl.ds(64 + 16 * k, 16)] = g3 * bb[3 + k]
                return 0
            lax.fori_loop(0, CH, edge_body, 0)

            pltpu.make_async_copy(ei.at[1, pl.ds(e0, CH)], dstss[p],
                                  sem_d[p]).wait()
            fire_scatter(p)
            fire_linear(i + 2, p)

        fire_linear(0, 0)
        fire_linear(1, 1)
        wait_linear(0)
        fire_gathers(0)

        run_chunk(0, 0, True)
        run_chunk(1, 1, True)

        def body2(k, _):
            run_chunk(2 * k, 0, False)
            run_chunk(2 * k + 1, 1, False)
            return 0
        lax.fori_loop(1, n_chunks // 2, body2, 0)

        wait_linear(1)
        wait_gathers(0)
        wait_scatter(0)
        wait_scatter(1)
        plsc.subcore_barrier()

        def drain(j, _):
            r0 = base_n + CH * j
            pltpu.sync_copy(acc.at[pl.ds(r0, CH)], pay0)
            pltpu.sync_copy(pay0.at[pl.ds(0, CH), pl.ds(0, 128)],
                            out.at[c, pl.ds(r0, CH)])
            pltpu.sync_copy(pay0.at[pl.ds(0, CH), pl.ds(128, 32)],
                            out2.at[c, pl.ds(r0, CH)])
            return 0
        lax.fori_loop(0, NPT // CH, drain, 0)
        if rem:
            r0 = base_n + (NPT // CH) * CH
            pltpu.sync_copy(acc.at[pl.ds(r0, rem)], pay0.at[pl.ds(0, rem)])
            pltpu.sync_copy(pay0.at[pl.ds(0, rem), pl.ds(0, 128)],
                            out.at[c, pl.ds(r0, rem)])
            pltpu.sync_copy(pay0.at[pl.ds(0, rem), pl.ds(128, 32)],
                            out2.at[c, pl.ds(r0, rem)])

    return sc_fn(edge_index, WB, U2, V2)


def _finish_kernel(ga0_ref, ga1_ref, gb0_ref, gb1_ref, lng_ref, lnb_ref,
                   ws0_ref, bs0_ref, ws1_ref, bs1_ref,
                   wa0_ref, wa1_ref, wb0_ref, wb1_ref, o_ref):
    ga = (ga0_ref[...], ga1_ref[...])
    gb = (gb0_ref[...], gb1_ref[...])

    def half_norm(a, b):
        q = a * a
        s = lambda k: q[:, 16 * k:16 * k + 16]
        qb = b * b
        trp = a[:, 64:80] + a[:, 80:96] + a[:, 96:112]
        return (3.0 * s(0) + 2.0 * (s(1) + s(2) + s(3))
                + s(4) + s(5) + s(6)
                + 2.0 * (s(7) + qb[:, 0:16] + qb[:, 16:32])
                - trp * trp / 3.0)

    nrm = jnp.concatenate([half_norm(ga[0], gb[0]),
                           half_norm(ga[1], gb[1])], axis=1)
    mu = jnp.mean(nrm, axis=1, keepdims=True)
    var = jnp.mean((nrm - mu) ** 2, axis=1, keepdims=True)
    nrm = (nrm - mu) / jnp.sqrt(var + 1e-5) * lng_ref[...] + lnb_ref[...]

    def mm_t(x, w):
        return lax.dot_general(x, w, (((1,), (1,)), ((), ())),
                               preferred_element_type=jnp.float32,
                               precision=lax.Precision.HIGHEST)

    h = mm_t(nrm, ws0_ref[...]) + bs0_ref[...]
    h = h * jax.nn.sigmoid(h)
    h = mm_t(h, ws1_ref[...]) + bs1_ref[...]
    h = h * jax.nn.sigmoid(h)
    n0 = h[:, 0:32]
    n1 = h[:, 32:64]
    n2 = h[:, 64:96]

    pb = (mm_t(ga[0], wa0_ref[...]) + mm_t(ga[1], wa1_ref[...]))
    pb2 = (mm_t(gb[0], wb0_ref[...]) + mm_t(gb[1], wb1_ref[...]))
    A0 = pb[:, 0:32]
    w0 = pb[:, 32:64]
    w1 = pb[:, 64:96]
    w2 = pb[:, 96:128]
    Pp = [pb[:, 128:160], pb[:, 160:192], pb[:, 192:224], pb[:, 224:256],
          pb2[:, 0:32], pb2[:, 32:64]]
    t3 = (Pp[0] + Pp[1] + Pp[2]) / 3.0

    diag = n0 * A0
    o_ref[0] = diag + n2 * (Pp[0] - t3)
    o_ref[1] = n2 * Pp[3] - n1 * w2
    o_ref[2] = n2 * Pp[5] + n1 * w1
    o_ref[3] = n2 * Pp[3] + n1 * w2
    o_ref[4] = diag + n2 * (Pp[1] - t3)
    o_ref[5] = n2 * Pp[4] - n1 * w0
    o_ref[6] = n2 * Pp[5] - n1 * w1
    o_ref[7] = n2 * Pp[4] + n1 * w0
    o_ref[8] = diag + n2 * (Pp[2] - t3)


def _node_finish(GhA, GhB, ln_g, ln_b, Ws0, bs0, Ws1, bs1, Wt0, Wt1, Wt2):
    perm = jnp.asarray([3 * c + k for k in range(3) for c in range(32)],
                       dtype=jnp.int32)
    ws1p = Ws1[perm, :]
    bs1p = bs1[perm]
    comps_a = [Wt0, Wt1, Wt1, Wt1, Wt2, Wt2, Wt2, Wt2]
    wa = [jnp.zeros((256, 128), jnp.float32) for _ in range(2)]
    wb = [jnp.zeros((64, 32), jnp.float32) for _ in range(2)]
    for h in range(2):
        for k, Wk in enumerate(comps_a):
            wa[h] = wa[h].at[32 * k:32 * k + 32, 16 * k:16 * k + 16].set(
                Wk[:, 16 * h:16 * h + 16])
        for k in range(2):
            wb[h] = wb[h].at[32 * k:32 * k + 32, 16 * k:16 * k + 16].set(
                Wt2[:, 16 * h:16 * h + 16])
    grid = (N_NODES // BN,)
    full = lambda s: pl.BlockSpec(s, lambda i: tuple(0 for _ in s))
    outs = pl.pallas_call(
        _finish_kernel,
        grid=grid,
        in_specs=[
            pl.BlockSpec((BN, 128), lambda i: (i, 0)),
            pl.BlockSpec((BN, 128), lambda i: (i, 0)),
            pl.BlockSpec((BN, 32), lambda i: (i, 0)),
            pl.BlockSpec((BN, 32), lambda i: (i, 0)),
            full((1, 32)), full((1, 32)), full((64, 32)), full((1, 64)),
            full((96, 64)), full((1, 96)),
            full((256, 128)), full((256, 128)), full((64, 32)), full((64, 32)),
        ],
        out_specs=[pl.BlockSpec((9, BN, 32), lambda i: (0, i, 0))],
        out_shape=[jax.ShapeDtypeStruct((9, N_NODES, 32), jnp.float32)],
    )(GhA[0], GhA[1], GhB[0], GhB[1], ln_g[None, :], ln_b[None, :],
      Ws0, bs0[None, :], ws1p, bs1p[None, :], wa[0], wa[1], wb[0], wb[1])
    return jnp.moveaxis(outs[0], 0, -1).reshape(N_NODES, UNITS, 3, 3)


def kernel(node_type, edge_index, edge_attr, bond_dist, bond_vec, emb_table,
           Wd1, bd1, Wd2, bd2, Wd3, bd3, W_emb2, b_emb2, W_emb3, b_emb3,
           Wt0, Wt1, Wt2, Ws0, bs0, Ws1, bs1, ln_g, ln_b):
    WB, edge_feat = _edge_precompute(
        edge_attr, bond_dist, bond_vec, Wd1, bd1, Wd2, bd2, Wd3, bd3,
        W_emb3, b_emb3)
    U2, V2 = _node_precompute(node_type, emb_table, W_emb2, b_emb2)
    GhA, GhB = _sc_scatter(edge_index, WB, U2, V2)
    X = _node_finish(GhA, GhB, ln_g, ln_b, Ws0, bs0, Ws1, bs1, Wt0, Wt1, Wt2)
    return X, edge_feat


# speedup vs baseline: 1.0194x; 1.0002x over previous
import functools

import jax
import jax.numpy as jnp
from jax import lax
from jax.experimental import pallas as pl
from jax.experimental.pallas import tpu as pltpu
from jax.experimental.pallas import tpu_sc as plsc

N_NODES = 10000
N_EDGES = 160000
UNITS = 32
CUTOFF = 5.0

NC = 2
NS = 16
CH = 40
EPW = N_EDGES // NS
NPT = N_NODES // NS
BE = 6400
BN = 1000


def _edge_kernel(ea_ref, bd_ref, bv_ref, wc_ref, bc_ref, we3_ref, be3_ref,
                 wb_ref, ef_ref):
    ea = ea_ref[...]

    def mm(x, w):
        return lax.dot_general(x, w, (((1,), (1,)), ((), ())),
                               preferred_element_type=jnp.float32,
                               precision=lax.Precision.HIGHEST)

    wb_ref[...] = mm(ea, wc_ref[...]) + bc_ref[...]

    r = bd_ref[...]
    c = jnp.where(r <= CUTOFF, 0.5 * (jnp.cos(jnp.pi * r / CUTOFF) + 1.0), 0.0)

    v = bv_ref[...]
    inv = 1.0 / jnp.sqrt(jnp.sum(v * v, axis=0, keepdims=True))
    ev = v * inv
    sq = ev * ev
    evr = jnp.concatenate([ev[1:], ev[:1]], axis=0)
    cr = ev * evr
    comp = jnp.concatenate([ev, sq, cr, c, ev, ev], axis=0)
    basc = comp.T

    wb_ref[:, 48:64] = basc
    wb_ref[:, 112:128] = basc

    ef_ref[...] = mm(ea, we3_ref[...]) + be3_ref[...]


def _edge_precompute(edge_attr, bond_dist, bond_vec, Wd1, bd1, Wd2, bd2,
                     Wd3, bd3, W_emb3, b_emb3):
    z16 = jnp.zeros((16, 32), jnp.float32)
    wcomb = jnp.concatenate(
        [Wd1[:16], Wd2[:16], Wd3[:16], z16,
         Wd1[16:], Wd2[16:], Wd3[16:], z16], axis=0)
    zb = jnp.zeros((16,), jnp.float32)
    bcomb = jnp.concatenate(
        [bd1[:16], bd2[:16], bd3[:16], zb,
         bd1[16:], bd2[16:], bd3[16:], zb], axis=0)
    grid = (N_EDGES // BE,)
    return pl.pallas_call(
        _edge_kernel,
        grid=grid,
        in_specs=[
            pl.BlockSpec((BE, 32), lambda i: (i, 0)),
            pl.BlockSpec((1, BE), lambda i: (0, i)),
            pl.BlockSpec((3, BE), lambda i: (0, i)),
            pl.BlockSpec((128, 32), lambda i: (0, 0)),
            pl.BlockSpec((1, 128), lambda i: (0, 0)),
            pl.BlockSpec((32, 32), lambda i: (0, 0)),
            pl.BlockSpec((1, 32), lambda i: (0, 0)),
        ],
        out_specs=[
            pl.BlockSpec((BE, 128), lambda i: (i, 0)),
            pl.BlockSpec((BE, 32), lambda i: (i, 0)),
        ],
        out_shape=[
            jax.ShapeDtypeStruct((N_EDGES, 128), jnp.float32),
            jax.ShapeDtypeStruct((N_EDGES, 32), jnp.float32),
        ],
    )(edge_attr, bond_dist[None, :], bond_vec.T, wcomb, bcomb[None, :],
      W_emb3, b_emb3[None, :])


def _node_kernel(nt_ref, emb_ref, wa_ref, wb_ref, b2_ref, u_ref, v_ref):
    nt = nt_ref[...]
    iota = lax.broadcasted_iota(jnp.int32, (BN, 128), 1)
    oh = (nt == iota).astype(jnp.float32)

    def mm_t(x, w):
        return lax.dot_general(x, w, (((1,), (1,)), ((), ())),
                               preferred_element_type=jnp.float32,
                               precision=lax.Precision.HIGHEST)

    def mm(x, w):
        return lax.dot_general(x, w, (((1,), (0,)), ((), ())),
                               preferred_element_type=jnp.float32,
                               precision=lax.Precision.HIGHEST)

    ma = mm_t(emb_ref[...], wa_ref[...])
    mb = mm_t(emb_ref[...], wb_ref[...])
    u = mm(oh, ma)
    v = mm(oh, mb) + b2_ref[...]
    u_ref[0] = u[:, :16]
    u_ref[1] = u[:, 16:]
    v_ref[0] = v[:, :16]
    v_ref[1] = v[:, 16:]


def _node_precompute(node_type, emb_table, W_emb2, b_emb2):
    emb_pad = jnp.zeros((128, 32), jnp.float32).at[:emb_table.shape[0]].set(
        emb_table)
    wa = W_emb2[:, :UNITS]
    wb = W_emb2[:, UNITS:]
    grid = (N_NODES // BN,)
    full = lambda s: pl.BlockSpec(s, lambda i: tuple(0 for _ in s))
    return pl.pallas_call(
        _node_kernel,
        grid=grid,
        in_specs=[
            pl.BlockSpec((BN, 1), lambda i: (i, 0)),
            full((128, 32)), full((32, 32)), full((32, 32)), full((1, 32)),
        ],
        out_specs=[
            pl.BlockSpec((2, BN, 16), lambda i: (0, i, 0)),
            pl.BlockSpec((2, BN, 16), lambda i: (0, i, 0)),
        ],
        out_shape=[
            jax.ShapeDtypeStruct((2, N_NODES, 16), jnp.float32),
            jax.ShapeDtypeStruct((2, N_NODES, 16), jnp.float32),
        ],
    )(node_type[:, None], emb_pad, wa, wb, b_emb2[None, :])


def _sc_scatter(edge_index, WB, U2, V2):
    mesh = plsc.VectorSubcoreMesh(core_axis_name="c", subcore_axis_name="s")
    n_chunks = EPW // CH

    vset = lambda: [pltpu.VMEM((CH,), jnp.int32),
                    pltpu.VMEM((CH,), jnp.int32),
                    pltpu.VMEM((CH, 64), jnp.float32),
                    pltpu.VMEM((CH, 16), jnp.float32),
                    pltpu.VMEM((CH, 16), jnp.float32)]

    @functools.partial(
        pl.kernel,
        out_type=[jax.ShapeDtypeStruct((NC, N_NODES, 128), jnp.float32),
                  jax.ShapeDtypeStruct((NC, N_NODES, 32), jnp.float32)],
        mesh=mesh,
        compiler_params=pltpu.CompilerParams(use_tc_tiling_on_sc=False),
        scratch_types=[
            pltpu.VMEM_SHARED((N_NODES, 160), jnp.float32),
            vset(), vset(),
            pltpu.VMEM((CH, 160), jnp.float32),
            pltpu.VMEM((CH, 160), jnp.float32),
            pltpu.VMEM((CH,), jnp.int32),
            pltpu.VMEM((CH,), jnp.int32),
            pltpu.SemaphoreType.DMA, pltpu.SemaphoreType.DMA,
            pltpu.SemaphoreType.DMA, pltpu.SemaphoreType.DMA,
            pltpu.SemaphoreType.DMA, pltpu.SemaphoreType.DMA,
            pltpu.SemaphoreType.DMA, pltpu.SemaphoreType.DMA,
        ],
    )
    def sc_fn(ei, wb, u2, v2, out, out2, acc, set0, set1, pay0, pay1,
              dsts0, dsts1, sl0, sl1, sg0, sg1, ss0, ss1, sd0, sd1):
        c = lax.axis_index("c")
        s = lax.axis_index("s")
        base_n = s * NPT
        sets = (set0, set1)
        pays = (pay0, pay1)
        dstss = (dsts0, dsts1)
        sem_l = (sl0, sl1)
        sem_g = (sg0, sg1)
        sem_s = (ss0, ss1)
        sem_d = (sd0, sd1)

        def zrow(i, _):
            for k in range(10):
                pay0[i, pl.ds(16 * k, 16)] = jnp.zeros((16,), jnp.float32)
            return 0
        lax.fori_loop(0, CH, zrow, 0)

        def zcopy(j, _):
            pltpu.sync_copy(pay0, acc.at[pl.ds(base_n + CH * j, CH)])
            return 0
        lax.fori_loop(0, NPT // CH, zcopy, 0)
        rem = NPT - (NPT // CH) * CH
        if rem:
            pltpu.sync_copy(pay0.at[pl.ds(0, rem)],
                            acc.at[pl.ds(base_n + (NPT // CH) * CH, rem)])
        plsc.subcore_barrier()

        def e_of(i):
            return s * EPW + jnp.minimum(i, n_chunks - 1) * CH

        def fire_linear(i, p):
            src_i, dst_i, wc, _, _ = sets[p]
            e0 = e_of(i)
            pltpu.async_copy(ei.at[0, pl.ds(e0, CH)], src_i, sem_l[p])
            pltpu.async_copy(ei.at[1, pl.ds(e0, CH)], dst_i, sem_l[p])
            pltpu.async_copy(wb.at[pl.ds(e0, CH), pl.ds(64 * c, 64)],
                             wc, sem_l[p])

        def wait_linear(p):
            src_i, dst_i, wc, _, _ = sets[p]
            e0 = s * EPW
            pltpu.make_async_copy(ei.at[0, pl.ds(e0, CH)], src_i, sem_l[p]).wait()
            pltpu.make_async_copy(ei.at[1, pl.ds(e0, CH)], dst_i, sem_l[p]).wait()
            pltpu.make_async_copy(wb.at[pl.ds(e0, CH), pl.ds(0, 64)],
                                  wc, sem_l[p]).wait()

        def fire_gathers(p):
            src_i, dst_i, _, ur, vr = sets[p]
            pltpu.async_copy(u2.at[c].at[src_i], ur, sem_g[p])
            pltpu.async_copy(v2.at[c].at[dst_i], vr, sem_g[p])

        def wait_gathers(p):
            src_i, dst_i, _, ur, vr = sets[p]
            pltpu.make_async_copy(u2.at[c].at[src_i], ur, sem_g[p]).wait()
            pltpu.make_async_copy(v2.at[c].at[dst_i], vr, sem_g[p]).wait()

        def fire_scatter(p):
            pltpu.async_copy(pays[p], acc.at[dstss[p]], sem_s[p], add=True)

        def wait_scatter(p):
            pltpu.make_async_copy(pays[p], acc.at[dstss[p]], sem_s[p]).wait()

        def run_chunk(i, p, first):
            q = 1 - p
            src_i, dst_i, wc, ur, vr = sets[p]
            pay = pays[p]
            wait_linear(q)
            fire_gathers(q)
            wait_gathers(p)
            if not first:
                wait_scatter(p)
            e0 = s * EPW + i * CH
            pltpu.async_copy(ei.at[1, pl.ds(e0, CH)], dstss[p], sem_d[p])

            def edge_body(e, _):
                bb = wc[e, pl.ds(48, 16)]
                zc = (ur[e, :] + vr[e, :]) * bb[9]
                g1 = zc * wc[e, pl.ds(0, 16)]
                g2 = zc * wc[e, pl.ds(16, 16)]
                g3 = zc * wc[e, pl.ds(32, 16)]
                pay[e, pl.ds(0, 16)] = g1
                for d in range(3):
                    pay[e, pl.ds(16 + 16 * d, 16)] = g2 * bb[d]
                for k in range(6):
                    pay[e, pl.ds(64 + 16 * k, 16)] = g3 * bb[3 + k]
                return 0
            lax.fori_loop(0, CH, edge_body, 0)

            pltpu.make_async_copy(ei.at[1, pl.ds(e0, CH)], dstss[p],
                                  sem_d[p]).wait()
            fire_scatter(p)
            fire_linear(i + 2, p)

        fire_linear(0, 0)
        fire_linear(1, 1)
        wait_linear(0)
        fire_gathers(0)

        run_chunk(0, 0, True)
        run_chunk(1, 1, True)

        def body2(k, _):
            run_chunk(2 * k, 0, False)
            run_chunk(2 * k + 1, 1, False)
            return 0
        lax.fori_loop(1, n_chunks // 2, body2, 0)

        wait_linear(1)
        wait_gathers(0)
        wait_scatter(0)
        wait_scatter(1)
        plsc.subcore_barrier()

        def drain(j, _):
            r0 = base_n + CH * j
            pltpu.sync_copy(acc.at[pl.ds(r0, CH)], pay0)
            pltpu.sync_copy(pay0.at[pl.ds(0, CH), pl.ds(0, 128)],
                            out.at[c, pl.ds(r0, CH)])
            pltpu.sync_copy(pay0.at[pl.ds(0, CH), pl.ds(128, 32)],
                            out2.at[c, pl.ds(r0, CH)])
            return 0
        lax.fori_loop(0, NPT // CH, drain, 0)
        if rem:
            r0 = base_n + (NPT // CH) * CH
            pltpu.sync_copy(acc.at[pl.ds(r0, rem)], pay0.at[pl.ds(0, rem)])
            pltpu.sync_copy(pay0.at[pl.ds(0, rem), pl.ds(0, 128)],
                            out.at[c, pl.ds(r0, rem)])
            pltpu.sync_copy(pay0.at[pl.ds(0, rem), pl.ds(128, 32)],
                            out2.at[c, pl.ds(r0, rem)])

    return sc_fn(edge_index, WB, U2, V2)


def _finish_kernel(ga0_ref, ga1_ref, gb0_ref, gb1_ref, lng_ref, lnb_ref,
                   ws0_ref, bs0_ref, ws1_ref, bs1_ref,
                   wa0_ref, wa1_ref, wb0_ref, wb1_ref, o_ref):
    ga = (ga0_ref[...], ga1_ref[...])
    gb = (gb0_ref[...], gb1_ref[...])

    def half_norm(a, b):
        q = a * a
        s = lambda k: q[:, 16 * k:16 * k + 16]
        qb = b * b
        trp = a[:, 64:80] + a[:, 80:96] + a[:, 96:112]
        return (3.0 * s(0) + 2.0 * (s(1) + s(2) + s(3))
                + s(4) + s(5) + s(6)
                + 2.0 * (s(7) + qb[:, 0:16] + qb[:, 16:32])
                - trp * trp / 3.0)

    nrm = jnp.concatenate([half_norm(ga[0], gb[0]),
                           half_norm(ga[1], gb[1])], axis=1)
    mu = jnp.mean(nrm, axis=1, keepdims=True)
    var = jnp.mean((nrm - mu) ** 2, axis=1, keepdims=True)
    nrm = (nrm - mu) / jnp.sqrt(var + 1e-5) * lng_ref[...] + lnb_ref[...]

    def mm_t(x, w):
        return lax.dot_general(x, w, (((1,), (1,)), ((), ())),
                               preferred_element_type=jnp.float32,
                               precision=lax.Precision.HIGHEST)

    h = mm_t(nrm, ws0_ref[...]) + bs0_ref[...]
    h = h * jax.nn.sigmoid(h)
    h = mm_t(h, ws1_ref[...]) + bs1_ref[...]
    h = h * jax.nn.sigmoid(h)
    n0 = h[:, 0:32]
    n1 = h[:, 32:64]
    n2 = h[:, 64:96]

    pb = (mm_t(ga[0], wa0_ref[...]) + mm_t(ga[1], wa1_ref[...]))
    pb2 = (mm_t(gb[0], wb0_ref[...]) + mm_t(gb[1], wb1_ref[...]))
    A0 = pb[:, 0:32]
    w0 = pb[:, 32:64]
    w1 = pb[:, 64:96]
    w2 = pb[:, 96:128]
    Pp = [pb[:, 128:160], pb[:, 160:192], pb[:, 192:224], pb[:, 224:256],
          pb2[:, 0:32], pb2[:, 32:64]]
    t3 = (Pp[0] + Pp[1] + Pp[2]) / 3.0

    diag = n0 * A0
    o_ref[0] = diag + n2 * (Pp[0] - t3)
    o_ref[1] = n2 * Pp[3] - n1 * w2
    o_ref[2] = n2 * Pp[5] + n1 * w1
    o_ref[3] = n2 * Pp[3] + n1 * w2
    o_ref[4] = diag + n2 * (Pp[1] - t3)
    o_ref[5] = n2 * Pp[4] - n1 * w0
    o_ref[6] = n2 * Pp[5] - n1 * w1
    o_ref[7] = n2 * Pp[4] + n1 * w0
    o_ref[8] = diag + n2 * (Pp[2] - t3)


def _node_finish(GhA, GhB, ln_g, ln_b, Ws0, bs0, Ws1, bs1, Wt0, Wt1, Wt2):
    perm = jnp.asarray([3 * c + k for k in range(3) for c in range(32)],
                       dtype=jnp.int32)
    ws1p = Ws1[perm, :]
    bs1p = bs1[perm]
    comps_a = [Wt0, Wt1, Wt1, Wt1, Wt2, Wt2, Wt2, Wt2]
    wa = [jnp.zeros((256, 128), jnp.float32) for _ in range(2)]
    wb = [jnp.zeros((64, 32), jnp.float32) for _ in range(2)]
    for h in range(2):
        for k, Wk in enumerate(comps_a):
            wa[h] = wa[h].at[32 * k:32 * k + 32, 16 * k:16 * k + 16].set(
                Wk[:, 16 * h:16 * h + 16])
        for k in range(2):
            wb[h] = wb[h].at[32 * k:32 * k + 32, 16 * k:16 * k + 16].set(
                Wt2[:, 16 * h:16 * h + 16])
    grid = (N_NODES // BN,)
    full = lambda s: pl.BlockSpec(s, lambda i: tuple(0 for _ in s))
    outs = pl.pallas_call(
        _finish_kernel,
        grid=grid,
        in_specs=[
            pl.BlockSpec((BN, 128), lambda i: (i, 0)),
            pl.BlockSpec((BN, 128), lambda i: (i, 0)),
            pl.BlockSpec((BN, 32), lambda i: (i, 0)),
            pl.BlockSpec((BN, 32), lambda i: (i, 0)),
            full((1, 32)), full((1, 32)), full((64, 32)), full((1, 64)),
            full((96, 64)), full((1, 96)),
            full((256, 128)), full((256, 128)), full((64, 32)), full((64, 32)),
        ],
        out_specs=[pl.BlockSpec((9, BN, 32), lambda i: (0, i, 0))],
        out_shape=[jax.ShapeDtypeStruct((9, N_NODES, 32), jnp.float32)],
    )(GhA[0], GhA[1], GhB[0], GhB[1], ln_g[None, :], ln_b[None, :],
      Ws0, bs0[None, :], ws1p, bs1p[None, :], wa[0], wa[1], wb[0], wb[1])
    return jnp.moveaxis(outs[0], 0, -1).reshape(N_NODES, UNITS, 3, 3)


def kernel(node_type, edge_index, edge_attr, bond_dist, bond_vec, emb_table,
           Wd1, bd1, Wd2, bd2, Wd3, bd3, W_emb2, b_emb2, W_emb3, b_emb3,
           Wt0, Wt1, Wt2, Ws0, bs0, Ws1, bs1, ln_g, ln_b):
    WB, edge_feat = _edge_precompute(
        edge_attr, bond_dist, bond_vec, Wd1, bd1, Wd2, bd2, Wd3, bd3,
        W_emb3, b_emb3)
    U2, V2 = _node_precompute(node_type, emb_table, W_emb2, b_emb2)
    GhA, GhB = _sc_scatter(edge_index, WB, U2, V2)
    X = _node_finish(GhA, GhB, ln_g, ln_b, Ws0, bs0, Ws1, bs1, Wt0, Wt1, Wt2)
    return X, edge_feat
